# baseline (speedup 1.0000x reference)
import functools

import jax
import jax.numpy as jnp
import numpy as np
from jax import lax
from jax.experimental import pallas as pl
from jax.experimental.pallas import tpu as pltpu
from jax.experimental.pallas import tpu_sc as plsc

SIZE = 256
CAMERA_RANGE = 3.0
AGENT_RADIUS = 0.05
OBSTACLE_BASE_HEIGHT = 0.5
HALF_FOV = float(np.radians(90.0)) / 2.0
NOBJ = 159
NPAD = 176
NTILES = 32
ROWS_PER_TILE = SIZE // NTILES
LANES = 16


def _paint_body(px_hbm, pr_hbm, pr2_hbm, dv_hbm, out_hbm,
                px_v, pr_v, pr2_v, dv_v, rows_v):
    cid = lax.axis_index("c")
    sid = lax.axis_index("s")
    wid = sid * 2 + cid

    pltpu.sync_copy(px_hbm, px_v)
    pltpu.sync_copy(pr_hbm, pr_v)
    pltpu.sync_copy(pr2_hbm, pr2_v)
    pltpu.sync_copy(dv_hbm, dv_v)

    ones = jnp.full((LANES,), 1.0, jnp.float32)
    for j in range(ROWS_PER_TILE):
        for sg in range(SIZE // LANES):
            rows_v[j, pl.ds(sg * LANES, LANES)] = ones

    iota = lax.convert_element_type(lax.iota(jnp.int32, LANES), jnp.float32)

    def obj_body(o, carry):
        px_o = px_v[pl.ds(o, LANES)][0]
        pr_o = pr_v[pl.ds(o, LANES)][0]
        pr2_o = pr2_v[pl.ds(o, LANES)][0]
        dv_o = dv_v[pl.ds(o, LANES)][0]
        clo = jnp.maximum(px_o - pr_o, 0.0)
        chi = jnp.minimum(px_o + pr_o, 255.0)
        vlo = lax.convert_element_type(clo, jnp.int32) // LANES
        vhi = lax.convert_element_type(chi, jnp.int32) // LANES
        for j in range(ROWS_PER_TILE):
            r = wid + NTILES * j
            dy = lax.convert_element_type(r, jnp.float32) - 128.0
            w2 = pr2_o - dy * dy

            @pl.when(w2 >= 0.0)
            def _paint_row():
                def seg_body(vb, c2):
                    base = vb * LANES
                    cols = lax.convert_element_type(base, jnp.float32) + iota
                    dx = cols - px_o
                    m = dx * dx <= w2
                    seg = rows_v[j, pl.ds(base, LANES)]
                    rows_v[j, pl.ds(base, LANES)] = jnp.where(
                        m, jnp.minimum(seg, dv_o), seg)
                    return c2

                lax.fori_loop(vlo, vhi + 1, seg_body, 0)

        return carry

    lax.fori_loop(0, NOBJ, obj_body, 0)

    for j in range(ROWS_PER_TILE):
        r = wid + NTILES * j
        pltpu.sync_copy(rows_v.at[j], out_hbm.at[r])


_paint = pl.kernel(
    _paint_body,
    out_type=jax.ShapeDtypeStruct((SIZE, SIZE), jnp.float32),
    mesh=plsc.VectorSubcoreMesh(core_axis_name="c", subcore_axis_name="s"),
    scratch_types=[
        pltpu.VMEM((NPAD,), jnp.float32),
        pltpu.VMEM((NPAD,), jnp.float32),
        pltpu.VMEM((NPAD,), jnp.float32),
        pltpu.VMEM((NPAD,), jnp.float32),
        pltpu.VMEM((ROWS_PER_TILE, SIZE), jnp.float32),
    ],
)


def kernel(agent_pos, goal_pos, other_agents, obstacles):
    vd = goal_pos - agent_pos
    vd = vd / (jnp.linalg.norm(vd) + 1e-08)
    cos_t = vd[1]
    sin_t = vd[0]
    Rm = jnp.stack([jnp.stack([cos_t, sin_t]), jnp.stack([-sin_t, cos_t])])

    def project_one(obj_pos, radius, height):
        rel = obj_pos - agent_pos
        fdot = jnp.dot(rel, Rm[1])
        cam = Rm @ rel
        dist = jnp.linalg.norm(cam)
        angle_x = jnp.arctan2(cam[0], cam[1])
        vis = (fdot >= 0.0) & (dist <= CAMERA_RANGE) & (jnp.abs(angle_x) <= HALF_FOV)
        pixel_x = angle_x / HALF_FOV * 0.5
        px_o = jnp.floor((pixel_x + 0.5) * SIZE)
        pr_o = jnp.floor(radius / (dist + 1e-08) * SIZE * 0.5)
        pr_o = jnp.clip(pr_o, 1.0, float(SIZE // 4))
        dv_o = jnp.minimum(dist / CAMERA_RANGE, 1.0)
        dv_o = dv_o * (1.0 - height * 0.3)
        dv_o = jnp.maximum(dv_o, 0.0)
        return px_o, pr_o, dv_o, vis

    per_obj = [project_one(other_agents[i], AGENT_RADIUS, 0.2)
               for i in range(other_agents.shape[0])]
    per_obj += [project_one(obstacles[i, :2], obstacles[i, 2], OBSTACLE_BASE_HEIGHT)
                for i in range(obstacles.shape[0])]
    px, pr, dval, visible = (jnp.stack(t) for t in zip(*per_obj))

    pad = NPAD - NOBJ
    px_k = jnp.pad(jnp.where(visible, px, 0.0), (0, pad))
    pr_k = jnp.pad(jnp.where(visible, pr, 0.0), (0, pad))
    pr2_k = jnp.pad(jnp.where(visible, pr * pr, -1.0), (0, pad),
                    constant_values=-1.0)
    dv_k = jnp.pad(jnp.where(visible, dval, 0.0), (0, pad))

    depth = _paint(px_k, pr_k, pr2_k, dv_k)
    return depth[None, :, :]


# speedup vs baseline: 16.5268x; 16.5268x over previous
import functools

import jax
import jax.numpy as jnp
import numpy as np
from jax import lax
from jax.experimental import pallas as pl
from jax.experimental.pallas import tpu as pltpu
from jax.experimental.pallas import tpu_sc as plsc

SIZE = 256
CAMERA_RANGE = 3.0
AGENT_RADIUS = 0.05
OBSTACLE_BASE_HEIGHT = 0.5
HALF_FOV = float(np.radians(90.0)) / 2.0
NOBJ = 159
NPAD = 176
NTILES = 32
ROWS_PER_TILE = SIZE // NTILES
LANES = 16


def _paint_body(px_hbm, pr_hbm, pr2_hbm, dv_hbm, out_hbm,
                px_v, pr_v, pr2_v, dv_v, rows_v):
    cid = lax.axis_index("c")
    sid = lax.axis_index("s")
    wid = sid * 2 + cid

    pltpu.sync_copy(px_hbm, px_v)
    pltpu.sync_copy(pr_hbm, pr_v)
    pltpu.sync_copy(pr2_hbm, pr2_v)
    pltpu.sync_copy(dv_hbm, dv_v)

    ones = jnp.full((LANES,), 1.0, jnp.float32)
    for j in range(ROWS_PER_TILE):
        for sg in range(SIZE // LANES):
            rows_v[j, pl.ds(sg * LANES, LANES)] = ones

    iota = lax.convert_element_type(lax.iota(jnp.int32, LANES), jnp.float32)

    def obj_body(o, carry):
        px_o = px_v[pl.ds(o, LANES)][0]
        pr_o = pr_v[pl.ds(o, LANES)][0]
        pr2_o = pr2_v[pl.ds(o, LANES)][0]
        dv_o = dv_v[pl.ds(o, LANES)][0]
        clo = jnp.maximum(px_o - pr_o, 0.0)
        chi = jnp.minimum(px_o + pr_o, 255.0)
        vlo = lax.convert_element_type(clo, jnp.int32) // LANES
        vhi = lax.convert_element_type(chi, jnp.int32) // LANES
        for j in range(ROWS_PER_TILE):
            r = wid + NTILES * j
            dy = lax.convert_element_type(r, jnp.float32) - 128.0
            w2 = pr2_o - dy * dy

            @pl.when(w2 >= 0.0)
            def _paint_row():
                def seg_body(vb, c2):
                    base = vb * LANES
                    cols = lax.convert_element_type(base, jnp.float32) + iota
                    dx = cols - px_o
                    m = dx * dx <= w2
                    seg = rows_v[j, pl.ds(base, LANES)]
                    rows_v[j, pl.ds(base, LANES)] = jnp.where(
                        m, jnp.minimum(seg, dv_o), seg)
                    return c2

                lax.fori_loop(vlo, vhi + 1, seg_body, 0)

        return carry

    lax.fori_loop(0, NOBJ, obj_body, 0)

    for j in range(ROWS_PER_TILE):
        r = wid + NTILES * j
        pltpu.sync_copy(rows_v.at[j], out_hbm.at[r])


_paint = pl.kernel(
    _paint_body,
    out_type=jax.ShapeDtypeStruct((SIZE, SIZE), jnp.float32),
    mesh=plsc.VectorSubcoreMesh(core_axis_name="c", subcore_axis_name="s"),
    scratch_types=[
        pltpu.VMEM((NPAD,), jnp.float32),
        pltpu.VMEM((NPAD,), jnp.float32),
        pltpu.VMEM((NPAD,), jnp.float32),
        pltpu.VMEM((NPAD,), jnp.float32),
        pltpu.VMEM((ROWS_PER_TILE, SIZE), jnp.float32),
    ],
)


def _bf16_rne(x):
    u = jax.lax.bitcast_convert_type(x, jnp.uint32)
    lsb = (u >> 16) & jnp.uint32(1)
    u = (u + jnp.uint32(0x7FFF) + lsb) & jnp.uint32(0xFFFF0000)
    return jax.lax.bitcast_convert_type(u, jnp.float32)


def kernel(agent_pos, goal_pos, other_agents, obstacles):
    vd = goal_pos - agent_pos
    vd = vd / (jnp.linalg.norm(vd) + 1e-08)
    cos_t = vd[1]
    sin_t = vd[0]

    pos = jnp.concatenate([other_agents, obstacles[:, :2]], axis=0)
    radius = jnp.concatenate([
        jnp.full((other_agents.shape[0],), AGENT_RADIUS, jnp.float32),
        obstacles[:, 2],
    ])
    height = jnp.concatenate([
        jnp.full((other_agents.shape[0],), 0.2, jnp.float32),
        jnp.full((obstacles.shape[0],), OBSTACLE_BASE_HEIGHT, jnp.float32),
    ])

    rel0 = pos[:, 0] - agent_pos[0]
    rel1 = pos[:, 1] - agent_pos[1]
    bc = _bf16_rne(jnp.broadcast_to(cos_t, rel0.shape))
    bs = _bf16_rne(jnp.broadcast_to(sin_t, rel0.shape))
    bns = _bf16_rne(jnp.broadcast_to(-sin_t, rel0.shape))
    br0 = _bf16_rne(rel0)
    br1 = _bf16_rne(rel1)
    cam0 = bc * br0 + bs * br1
    cam1 = bns * br0 + bc * br1
    fdot = rel0 * (-sin_t) + rel1 * cos_t
    dist = jnp.sqrt(cam0 * cam0 + cam1 * cam1)
    angle_x = jnp.arctan2(cam0, cam1)
    visible = (fdot >= 0.0) & (dist <= CAMERA_RANGE) & (jnp.abs(angle_x) <= HALF_FOV)
    pixel_x = angle_x / HALF_FOV * 0.5
    px = jnp.floor((pixel_x + 0.5) * SIZE)
    pr = jnp.floor(radius / (dist + 1e-08) * SIZE * 0.5)
    pr = jnp.clip(pr, 1.0, float(SIZE // 4))
    dval = jnp.minimum(dist / CAMERA_RANGE, 1.0)
    dval = dval * (1.0 - height * 0.3)
    dval = jnp.maximum(dval, 0.0)

    pad = NPAD - NOBJ
    px_k = jnp.pad(jnp.where(visible, px, 0.0), (0, pad))
    pr_k = jnp.pad(jnp.where(visible, pr, 0.0), (0, pad))
    pr2_k = jnp.pad(jnp.where(visible, pr * pr, -1.0), (0, pad),
                    constant_values=-1.0)
    dv_k = jnp.pad(jnp.where(visible, dval, 0.0), (0, pad))

    depth = _paint(px_k, pr_k, pr2_k, dv_k)
    return depth[None, :, :]
